# baseline (speedup 1.0000x reference)
import functools

import jax
import jax.numpy as jnp
from jax import lax
from jax.experimental import pallas as pl
from jax.experimental.pallas import tpu as pltpu
from jax.experimental.pallas import tpu_sc as plsc

_D = 32
_C = 1280
_S = 4
_CS = _C // _S


@functools.cache
def _make_lookup(B: int, V: int):
    info = plsc.get_sparse_core_info()
    nc, ns = info.num_cores, info.num_subcores
    nw = nc * ns
    b_per_w = B // nw
    n_chunks = b_per_w // _C
    assert b_per_w % _C == 0 and n_chunks % 2 == 0

    mesh = plsc.VectorSubcoreMesh(core_axis_name="c", subcore_axis_name="s")

    @functools.partial(
        pl.kernel,
        mesh=mesh,
        compiler_params=pltpu.CompilerParams(use_tc_tiling_on_sc=False),
        out_type=jax.ShapeDtypeStruct((B, _D), jnp.float32),
        scratch_types=[
            pltpu.VMEM((_C,), jnp.int32),
            pltpu.VMEM((_C,), jnp.int32),
            pltpu.VMEM((_C, _D), jnp.float32),
            pltpu.VMEM((_C, _D), jnp.float32),
            pltpu.SemaphoreType.DMA,
            pltpu.SemaphoreType.DMA,
            pltpu.SemaphoreType.DMA,
            pltpu.SemaphoreType.DMA,
            pltpu.SemaphoreType.DMA,
            pltpu.SemaphoreType.DMA,
        ],
    )
    def lookup(table_hbm, idx_hbm, out_hbm,
               idx0, idx1, rows0, rows1,
               si0, si1, sg0, sg1, so0, so1):
        wid = lax.axis_index("s") * nc + lax.axis_index("c")
        base = wid * b_per_w
        idx_b = (idx0, idx1)
        rows_b = (rows0, rows1)
        si = (si0, si1)
        sg = (sg0, sg1)
        so = (so0, so1)

        def row0_of(ci):
            return pl.multiple_of(base + ci * _C, 8)

        pltpu.async_copy(idx_hbm.at[pl.ds(row0_of(0), _C)], idx0, si0)

        def stage(ci, p):
            q = 1 - p

            @pl.when(ci >= 2)
            def _():
                pltpu.make_async_copy(
                    rows_b[p], out_hbm.at[pl.ds(row0_of(ci - 2), _C)], so[p]
                ).wait()

            pltpu.make_async_copy(
                idx_hbm.at[pl.ds(row0_of(ci), _C)], idx_b[p], si[p]
            ).wait()
            for s in range(_S):
                pltpu.async_copy(
                    table_hbm.at[idx_b[p].at[pl.ds(s * _CS, _CS)]],
                    rows_b[p].at[pl.ds(s * _CS, _CS)],
                    sg[p],
                )

            @pl.when(ci >= 1)
            def _():
                for s in range(_S):
                    pltpu.make_async_copy(
                        table_hbm.at[idx_b[q].at[pl.ds(s * _CS, _CS)]],
                        rows_b[q].at[pl.ds(s * _CS, _CS)],
                        sg[q],
                    ).wait()
                pltpu.async_copy(
                    rows_b[q], out_hbm.at[pl.ds(row0_of(ci - 1), _C)], so[q]
                )

            @pl.when(ci + 1 < n_chunks)
            def _():
                pltpu.async_copy(
                    idx_hbm.at[pl.ds(row0_of(ci + 1), _C)], idx_b[q], si[q]
                )

        @pl.loop(0, n_chunks, step=2)
        def _pair(ci0):
            stage(ci0, 0)
            stage(ci0 + 1, 1)

        last = n_chunks - 1
        for s in range(_S):
            pltpu.make_async_copy(
                table_hbm.at[idx1.at[pl.ds(s * _CS, _CS)]],
                rows1.at[pl.ds(s * _CS, _CS)],
                sg1,
            ).wait()
        pltpu.async_copy(rows1, out_hbm.at[pl.ds(row0_of(last), _C)], so1)
        pltpu.make_async_copy(
            rows0, out_hbm.at[pl.ds(row0_of(last - 1), _C)], so0
        ).wait()
        pltpu.make_async_copy(
            rows1, out_hbm.at[pl.ds(row0_of(last), _C)], so1
        ).wait()

    return lookup


def kernel(indices, weight):
    B = indices.size
    idx_flat = indices.reshape(B).astype(jnp.int32)
    out = _make_lookup(B, weight.shape[0])(weight, idx_flat)
    return out.reshape(indices.shape + (weight.shape[1],))


# speedup vs baseline: 1.6318x; 1.6318x over previous
import functools

import jax
import jax.numpy as jnp
from jax import lax
from jax.experimental import pallas as pl
from jax.experimental.pallas import tpu as pltpu
from jax.experimental.pallas import tpu_sc as plsc

_D = 32
_T = 50
_C = 1600
_CT = _C // _T
_S = 4
_CS = _C // _S


@functools.cache
def _make_lookup(NT: int, V: int):
    B = NT * _T
    info = plsc.get_sparse_core_info()
    nc, ns = info.num_cores, info.num_subcores
    nw = nc * ns
    b_per_w = B // nw
    t_per_w = NT // nw
    n_chunks = b_per_w // _C
    assert b_per_w % _C == 0 and n_chunks % 2 == 0

    mesh = plsc.VectorSubcoreMesh(core_axis_name="c", subcore_axis_name="s")

    @functools.partial(
        pl.kernel,
        mesh=mesh,
        compiler_params=pltpu.CompilerParams(use_tc_tiling_on_sc=False),
        out_type=jax.ShapeDtypeStruct((NT, _T, _D), jnp.float32),
        scratch_types=[
            pltpu.VMEM((_C,), jnp.int32),
            pltpu.VMEM((_C,), jnp.int32),
            pltpu.VMEM((_C, _D), jnp.float32),
            pltpu.VMEM((_C, _D), jnp.float32),
            pltpu.SemaphoreType.DMA,
            pltpu.SemaphoreType.DMA,
            pltpu.SemaphoreType.DMA,
            pltpu.SemaphoreType.DMA,
            pltpu.SemaphoreType.DMA,
            pltpu.SemaphoreType.DMA,
        ],
    )
    def lookup(table_hbm, idx_hbm, out_hbm,
               idx0, idx1, rows0, rows1,
               si0, si1, sg0, sg1, so0, so1):
        wid = lax.axis_index("s") * nc + lax.axis_index("c")
        base = wid * b_per_w
        base_tok = wid * t_per_w
        idx_b = (idx0, idx1)
        rows_b = (rows0, rows1)
        si = (si0, si1)
        sg = (sg0, sg1)
        so = (so0, so1)

        def row0_of(ci):
            return pl.multiple_of(base + ci * _C, 8)

        def store_chunk(ci, p, wait):
            tok0 = base_tok + ci * _CT
            for t in range(_CT):
                cp = pltpu.make_async_copy(
                    rows_b[p].at[pl.ds(t * _T, _T)], out_hbm.at[tok0 + t], so[p]
                )
                if wait:
                    cp.wait()
                else:
                    cp.start()

        pltpu.async_copy(idx_hbm.at[pl.ds(row0_of(0), _C)], idx0, si0)

        def stage(ci, p):
            q = 1 - p

            @pl.when(ci >= 2)
            def _():
                store_chunk(ci - 2, p, wait=True)

            pltpu.make_async_copy(
                idx_hbm.at[pl.ds(row0_of(ci), _C)], idx_b[p], si[p]
            ).wait()
            for s in range(_S):
                pltpu.async_copy(
                    table_hbm.at[idx_b[p].at[pl.ds(s * _CS, _CS)]],
                    rows_b[p].at[pl.ds(s * _CS, _CS)],
                    sg[p],
                )

            @pl.when(ci >= 1)
            def _():
                for s in range(_S):
                    pltpu.make_async_copy(
                        table_hbm.at[idx_b[q].at[pl.ds(s * _CS, _CS)]],
                        rows_b[q].at[pl.ds(s * _CS, _CS)],
                        sg[q],
                    ).wait()
                store_chunk(ci - 1, q, wait=False)

            @pl.when(ci + 1 < n_chunks)
            def _():
                pltpu.async_copy(
                    idx_hbm.at[pl.ds(row0_of(ci + 1), _C)], idx_b[q], si[q]
                )

        @pl.loop(0, n_chunks, step=2)
        def _pair(ci0):
            stage(ci0, 0)
            stage(ci0 + 1, 1)

        last = n_chunks - 1
        for s in range(_S):
            pltpu.make_async_copy(
                table_hbm.at[idx1.at[pl.ds(s * _CS, _CS)]],
                rows1.at[pl.ds(s * _CS, _CS)],
                sg1,
            ).wait()
        store_chunk(last, 1, wait=False)
        store_chunk(last - 1, 0, wait=True)
        store_chunk(last, 1, wait=True)

    return lookup


def kernel(indices, weight):
    NT = indices.shape[0]
    B = indices.size
    idx_flat = indices.reshape(B).astype(jnp.int32)
    return _make_lookup(NT, weight.shape[0])(weight, idx_flat)


# speedup vs baseline: 1.6326x; 1.0005x over previous
import functools

import jax
import jax.numpy as jnp
from jax import lax
from jax.experimental import pallas as pl
from jax.experimental.pallas import tpu as pltpu
from jax.experimental.pallas import tpu_sc as plsc

_D = 32
_T = 50
_C = 1600
_CT = _C // _T
_S = 4
_CS = _C // _S


@functools.cache
def _make_lookup(NT: int, V: int):
    B = NT * _T
    info = plsc.get_sparse_core_info()
    nc, ns = info.num_cores, info.num_subcores
    nw = nc * ns
    b_per_w = B // nw
    t_per_w = NT // nw
    n_chunks = b_per_w // _C
    assert b_per_w % _C == 0 and n_chunks % 2 == 0

    mesh = plsc.VectorSubcoreMesh(core_axis_name="c", subcore_axis_name="s")

    @functools.partial(
        pl.kernel,
        mesh=mesh,
        compiler_params=pltpu.CompilerParams(use_tc_tiling_on_sc=False),
        out_type=jax.ShapeDtypeStruct((NT, _T, _D), jnp.float32),
        scratch_types=[
            pltpu.VMEM((_C,), jnp.int32),
            pltpu.VMEM((_C,), jnp.int32),
            pltpu.VMEM((_C, _D), jnp.float32),
            pltpu.VMEM((_C, _D), jnp.float32),
            pltpu.SemaphoreType.DMA,
            pltpu.SemaphoreType.DMA,
            pltpu.SemaphoreType.DMA,
            pltpu.SemaphoreType.DMA,
            pltpu.SemaphoreType.DMA,
            pltpu.SemaphoreType.DMA,
        ],
    )
    def lookup(table_hbm, idx_hbm, out_hbm,
               idx0, idx1, rows0, rows1,
               si0, si1, sg0, sg1, so0, so1):
        wid = lax.axis_index("s") * nc + lax.axis_index("c")
        base = wid * b_per_w
        base_tok = wid * t_per_w
        idx_b = (idx0, idx1)
        rows_b = (rows0, rows1)
        si = (si0, si1)
        sg = (sg0, sg1)
        so = (so0, so1)

        def row0_of(ci):
            return pl.multiple_of(base + ci * _C, 8)

        def store_chunk(ci, p, wait):
            tok0 = base_tok + ci * _CT
            for t in range(_CT):
                cp = pltpu.make_async_copy(
                    rows_b[p].at[pl.ds(t * _T, _T)], out_hbm.at[tok0 + t], so[p]
                )
                if wait:
                    cp.wait()
                else:
                    cp.start()

        pltpu.async_copy(idx_hbm.at[pl.ds(row0_of(0), _C)], idx0, si0)

        def stage(ci, p):
            q = 1 - p

            @pl.when(ci >= 2)
            def _():
                store_chunk(ci - 2, p, wait=True)

            pltpu.make_async_copy(
                idx_hbm.at[pl.ds(row0_of(ci), _C)], idx_b[p], si[p]
            ).wait()
            for s in range(_S):
                pltpu.async_copy(
                    table_hbm.at[idx_b[p].at[pl.ds(s * _CS, _CS)]],
                    rows_b[p].at[pl.ds(s * _CS, _CS)],
                    sg[p],
                )

            @pl.when(ci >= 1)
            def _():
                for s in range(_S):
                    pltpu.make_async_copy(
                        table_hbm.at[idx_b[q].at[pl.ds(s * _CS, _CS)]],
                        rows_b[q].at[pl.ds(s * _CS, _CS)],
                        sg[q],
                    ).wait()
                store_chunk(ci - 1, q, wait=False)

            @pl.when(ci + 1 < n_chunks)
            def _():
                pltpu.async_copy(
                    idx_hbm.at[pl.ds(row0_of(ci + 1), _C)], idx_b[q], si[q]
                )

        @pl.loop(0, n_chunks, step=2)
        def _pair(ci0):
            stage(ci0, 0)
            stage(ci0 + 1, 1)

        last = n_chunks - 1
        for s in range(_S):
            pltpu.make_async_copy(
                table_hbm.at[idx1.at[pl.ds(s * _CS, _CS)]],
                rows1.at[pl.ds(s * _CS, _CS)],
                sg1,
            ).wait()
        store_chunk(last, 1, wait=False)
        store_chunk(last - 1, 0, wait=True)
        store_chunk(last, 1, wait=True)

    return lookup


def kernel(indices, weight):
    NT = indices.shape[0]
    B = indices.size
    idx_flat = indices.reshape(B).astype(jnp.int32)
    w_lin = lax.optimization_barrier(weight.reshape(weight.size))
    w2 = w_lin.reshape(weight.shape)
    return _make_lookup(NT, weight.shape[0])(w2, idx_flat)
